# baseline (speedup 1.0000x reference)
import functools
import math

import jax
import jax.numpy as jnp
from jax import lax
from jax.experimental import pallas as pl
from jax.experimental.pallas import tpu as pltpu
from jax.experimental.pallas import tpu_sc as plsc

_D = 64
_SCALE = math.sqrt(_D)
_LANES = 16
_CHUNK = 128
_NBUF = 4
_AHEAD = 2


def _embed_sc(x2d, tab2):
    n_rows = x2d.shape[0]
    B = n_rows * _CHUNK
    info = plsc.get_sparse_core_info()
    nw = info.num_cores * info.num_subcores
    per_w = B // nw
    n_chunks = per_w // _CHUNK
    assert per_w % _CHUNK == 0 and n_chunks % _NBUF == 0 and n_chunks >= 2 * _NBUF

    mesh = plsc.VectorSubcoreMesh(core_axis_name="c", subcore_axis_name="s")

    @functools.partial(
        pl.kernel,
        out_type=jax.ShapeDtypeStruct((B, 2 * _D), jnp.float32),
        mesh=mesh,
        scratch_types=[
            pltpu.VMEM((_NBUF, _CHUNK), jnp.int32),
            pltpu.VMEM((_NBUF, _CHUNK), jnp.int32),
            pltpu.VMEM((_NBUF, _CHUNK, 2 * _D), jnp.float32),
            pltpu.SemaphoreType.DMA((_NBUF,)),
            pltpu.SemaphoreType.DMA((_NBUF,)),
        ],
        compiler_params=pltpu.CompilerParams(use_tc_tiling_on_sc=True),
    )
    def k(x_hbm, tab_hbm, out_hbm, idx_v, pr_v, rows_v, sem_g, sem_o):
        wid = lax.axis_index("s") * info.num_cores + lax.axis_index("c")
        row_base_w = wid * n_chunks

        def fire_gather(g, b):
            pltpu.sync_copy(x_hbm.at[row_base_w + g], idx_v.at[b])
            for i in range(_CHUNK // _LANES):
                sl = pl.ds(i * _LANES, _LANES)
                pr_v[b, sl] = idx_v[b, sl] >> 1
            pltpu.async_copy(
                tab_hbm.at[pr_v.at[b]], rows_v.at[b], sem_g.at[b]
            )

        def drain_gather(b):
            pltpu.make_async_copy(
                tab_hbm.at[pl.ds(0, _CHUNK)], rows_v.at[b], sem_g.at[b]
            ).wait()

        def select_scale(b):
            def grp(i, c):
                par = (idx_v[b, pl.ds(i * _LANES, _LANES)] & 1) * _D
                for dr in range(_LANES):
                    off = par[dr]
                    r = i * _LANES + dr
                    for j in range(_D // _LANES):
                        rows_v[b, r, pl.ds(j * _LANES, _LANES)] = (
                            rows_v[b, r, pl.ds(off + j * _LANES, _LANES)]
                            * _SCALE
                        )
                return c

            lax.fori_loop(0, _CHUNK // _LANES, grp, 0, unroll=False)

        def fire_out(g, b):
            pltpu.async_copy(
                rows_v.at[b],
                out_hbm.at[pl.ds((row_base_w + g) * _CHUNK, _CHUNK)],
                sem_o.at[b],
            )

        def drain_out(b):
            pltpu.make_async_copy(
                rows_v.at[b], out_hbm.at[pl.ds(0, _CHUNK)], sem_o.at[b]
            ).wait()

        for g in range(_AHEAD):
            fire_gather(g, g)

        for g in range(_AHEAD):
            b = g % _NBUF
            drain_gather(b)
            select_scale(b)
            fire_out(g, b)
            fire_gather(g + _AHEAD, (g + _AHEAD) % _NBUF)

        n_steady = n_chunks - 2 * _AHEAD
        assert n_steady % _NBUF == 0

        def super_step(s, carry):
            for p in range(_NBUF):
                g = _AHEAD + s * _NBUF + p
                b = (_AHEAD + p) % _NBUF
                drain_gather(b)
                select_scale(b)
                fire_out(g, b)
                drain_out(p)
                fire_gather(g + _AHEAD, p)
            return carry

        lax.fori_loop(0, n_steady // _NBUF, super_step, 0, unroll=False)

        for g in range(n_chunks - _AHEAD, n_chunks):
            b = g % _NBUF
            drain_gather(b)
            select_scale(b)
            fire_out(g, b)

        for b in range(_NBUF):
            drain_out(b)

    return k(x2d, tab2)


def kernel(x, table):
    b, t = x.shape
    x2d = x.reshape(-1, _CHUNK).astype(jnp.int32)
    tab2 = table.reshape(-1, 2 * _D)
    out = _embed_sc(x2d, tab2)
    return out[:, :_D].reshape(b, t, _D)


# speedup vs baseline: 1.4896x; 1.4896x over previous
import functools
import math

import jax
import jax.numpy as jnp
from jax import lax
from jax.experimental import pallas as pl
from jax.experimental.pallas import tpu as pltpu
from jax.experimental.pallas import tpu_sc as plsc

_D = 64
_SCALE = math.sqrt(_D)
_LANES = 16
_IDX_W = 128
_CHUNK = 256
_K = _CHUNK // _IDX_W
_NBUF = 4
_AHEAD = 2


def _embed_sc(x2d, table):
    n_rows = x2d.shape[0]
    B = n_rows * _IDX_W
    info = plsc.get_sparse_core_info()
    nw = info.num_cores * info.num_subcores
    per_w = B // nw
    n_chunks = per_w // _CHUNK
    rpc = _CHUNK // _IDX_W
    assert per_w % _CHUNK == 0 and n_chunks % _NBUF == 0 and n_chunks >= 2 * _NBUF

    mesh = plsc.VectorSubcoreMesh(core_axis_name="c", subcore_axis_name="s")

    @functools.partial(
        pl.kernel,
        out_type=jax.ShapeDtypeStruct((B, 2 * _D), jnp.float32),
        mesh=mesh,
        scratch_types=[
            pltpu.VMEM((_NBUF, _K, _IDX_W), jnp.int32),
            pltpu.VMEM((_NBUF, _CHUNK, _D), jnp.float32),
            pltpu.SemaphoreType.DMA((_NBUF,)),
            pltpu.SemaphoreType.DMA((_NBUF,)),
        ],
        compiler_params=pltpu.CompilerParams(use_tc_tiling_on_sc=False),
    )
    def k(x_hbm, tab_hbm, out_hbm, idx_v, rows_v, sem_g, sem_o):
        wid = lax.axis_index("s") * info.num_cores + lax.axis_index("c")
        row_base_w = wid * (per_w // _IDX_W)

        def fire_gather(g, b):
            pltpu.sync_copy(
                x_hbm.at[pl.ds(row_base_w + g * rpc, rpc)], idx_v.at[b]
            )
            for j in range(_K):
                pltpu.async_copy(
                    tab_hbm.at[idx_v.at[b, j]],
                    rows_v.at[b, pl.ds(j * _IDX_W, _IDX_W)],
                    sem_g.at[b],
                )

        def drain_gather(b):
            pltpu.make_async_copy(
                tab_hbm.at[pl.ds(0, _CHUNK)], rows_v.at[b], sem_g.at[b]
            ).wait()

        def scale(b):
            def body(i, c):
                r = i * 4
                for dr in range(4):
                    for j in range(_D // _LANES):
                        sl = (b, r + dr, pl.ds(j * _LANES, _LANES))
                        rows_v[sl] = rows_v[sl] * _SCALE
                return c

            lax.fori_loop(0, _CHUNK // 4, body, 0, unroll=False)

        def fire_out(g, b):
            pltpu.async_copy(
                rows_v.at[b],
                out_hbm.at[
                    pl.ds((row_base_w + g * rpc) * _IDX_W, _CHUNK),
                    pl.ds(0, _D),
                ],
                sem_o.at[b],
            )

        def drain_out(b):
            pltpu.make_async_copy(
                rows_v.at[b],
                out_hbm.at[pl.ds(0, _CHUNK), pl.ds(0, _D)],
                sem_o.at[b],
            ).wait()

        for g in range(_AHEAD):
            fire_gather(g, g)

        for g in range(_AHEAD):
            b = g % _NBUF
            drain_gather(b)
            scale(b)
            fire_out(g, b)
            fire_gather(g + _AHEAD, (g + _AHEAD) % _NBUF)

        n_steady = n_chunks - 2 * _AHEAD
        assert n_steady % _NBUF == 0

        def super_step(s, carry):
            for p in range(_NBUF):
                g = _AHEAD + s * _NBUF + p
                b = (_AHEAD + p) % _NBUF
                drain_gather(b)
                scale(b)
                fire_out(g, b)
                drain_out(p)
                fire_gather(g + _AHEAD, p)
            return carry

        lax.fori_loop(0, n_steady // _NBUF, super_step, 0, unroll=False)

        for g in range(n_chunks - _AHEAD, n_chunks):
            b = g % _NBUF
            drain_gather(b)
            scale(b)
            fire_out(g, b)

        for b in range(_NBUF):
            drain_out(b)

    return k(x2d, table)


def kernel(x, table):
    b, t = x.shape
    x2d = x.reshape(-1, _IDX_W).astype(jnp.int32)
    out = _embed_sc(x2d, table)
    return out[:, :_D].reshape(b, t, _D)
